# baseline (speedup 1.0000x reference)
import functools

import jax
import jax.numpy as jnp
from jax import lax
from jax.experimental import pallas as pl
from jax.experimental.pallas import tpu as pltpu
from jax.experimental.pallas import tpu_sc as plsc

D = 64
SCALE = 8.0

_NC = 2
_NS = 16
_NW = _NC * _NS

_NI = 4096
_NJ = 200
_V = 1000000
_KV = (_V + 127) // 128
_VP = _KV * 128
_FLAGF = 1325.0625

_I_PER_W = _NI // _NW


def _make_kernel():
    mesh = plsc.VectorSubcoreMesh(core_axis_name="c", subcore_axis_name="s")

    @functools.partial(
        pl.kernel,
        mesh=mesh,
        out_type=(
            jax.ShapeDtypeStruct((_NI, _NJ, D), jnp.float32),
            jax.ShapeDtypeStruct((_VP + _NW, 128), jnp.float32),
        ),
        scratch_types=[
            pltpu.VMEM((64, 129), jnp.float32),
            pltpu.VMEM((64, 129), jnp.float32),
            pltpu.VMEM((64, 128), jnp.float32),
            pltpu.VMEM((64, 128), jnp.float32),
            pltpu.VMEM((_NJ, 128), jnp.float32),
            pltpu.VMEM((_NJ, 128), jnp.float32),
            pltpu.VMEM((_NJ, D), jnp.float32),
            pltpu.VMEM((256,), jnp.int32),
            pltpu.VMEM((256,), jnp.int32),
            pltpu.VMEM((_NW, 128), jnp.float32),
            pltpu.VMEM((128,), jnp.float32),
            pltpu.VMEM((272,), jnp.float32),
            pltpu.VMEM((272,), jnp.float32),
            pltpu.SemaphoreType.DMA,
            pltpu.SemaphoreType.DMA,
            pltpu.SemaphoreType.DMA,
            pltpu.SemaphoreType.DMA,
            pltpu.SemaphoreType.DMA,
            pltpu.SemaphoreType.DMA,
            pltpu.SemaphoreType.DMA,
            pltpu.SemaphoreType.DMA,
            pltpu.SemaphoreType.DMA,
        ],
        compiler_params=pltpu.CompilerParams(
            use_tc_tiling_on_sc=True, needs_layout_passes=False),
    )
    def k(idx_hbm, tabt_hbm, out_hbm, stage_hbm,
          tbuf0, tbuf1, trow0, trow1, rows0, rows1, slab, idxv0, idxv1,
          flagv, fbuf, bnc0, bnc1, ts0, ts1, ws0, ws1, gs0, gs1, is0, is1,
          wsem):
        wid = lax.axis_index("s") * _NC + lax.axis_index("c")
        tb = (tbuf0, tbuf1)
        tr = (trow0, trow1)
        tsem = (ts0, ts1)
        wsm = (ws0, ws1)

        for t in range(8):
            fbuf[pl.ds(t * 16, 16)] = jnp.zeros((16,), jnp.float32)
        pltpu.sync_copy(fbuf, stage_hbm.at[_VP + wid])

        kv0 = (wid * _KV) // _NW
        kv1 = ((wid + 1) * _KV) // _NW

        def fire_tiles(kv, b):
            for kd in range(8):
                pltpu.async_copy(
                    tabt_hbm.at[pl.ds(kd * 8, 8), pl.ds(kv * 128, 128)],
                    tb[b].at[pl.ds(kd * 8, 8), pl.ds(0, 128)],
                    tsem[b],
                )

        def wait_tiles(b):
            pltpu.make_async_copy(
                stage_hbm.at[pl.ds(0, 64)],
                tb[b].at[pl.ds(0, 64), pl.ds(0, 128)], tsem[b]).wait()

        def wait_stage_wb(h):
            pltpu.make_async_copy(
                tr[h], stage_hbm.at[pl.ds(0, 64)], wsm[h]).wait()

        def pcol(kv, b):
            @pl.when(kv + 1 < kv1)
            def _():
                fire_tiles(kv + 1, 1 - b)

            wait_tiles(b)
            iota17 = lax.iota(jnp.int32, 16) * 17

            for h in range(2):
                @pl.when(kv - kv0 >= 1)
                def _():
                    wait_stage_wb(h)

                def trans(v4, c2):
                    vv = v4 * 4 + h * 64
                    for u in range(4):
                        col = jnp.full((16,), vv + u, jnp.int32)
                        for db in range(4):
                            rowi = lax.iota(jnp.int32, 16) + db * 16
                            vals = plsc.load_gather(tb[b], [rowi, col])
                            tr[h][v4 * 4 + u, pl.ds(db * 16, 16)] = vals
                    return c2

                lax.fori_loop(0, 16, trans, 0)
                pltpu.async_copy(
                    tr[h], stage_hbm.at[pl.ds(kv * 128 + h * 64, 64)],
                    wsm[h])

        fire_tiles(kv0, 0)

        def pstep(s, carry):
            for par in range(2):
                kv = kv0 + s * 2 + par

                @pl.when(kv < kv1)
                def _():
                    pcol(kv, par)

            return carry

        lax.fori_loop(0, (_KV // _NW) // 2 + 1, pstep, 0)

        wait_stage_wb(0)
        wait_stage_wb(1)

        for t in range(8):
            fbuf[pl.ds(t * 16, 16)] = jnp.full((16,), _FLAGF, jnp.float32)
        pltpu.sync_copy(fbuf, stage_hbm.at[_VP + wid])

        def cond(c):
            return c != _NW * 16

        def poll(c):
            pltpu.sync_copy(stage_hbm.at[pl.ds(_VP, _NW)], flagv)
            n = jnp.zeros((16,), jnp.int32)
            for t in range(_NW):
                n = n + jnp.where(flagv[t, pl.ds(0, 16)] == _FLAGF, 1, 0)
            return jnp.sum(n)

        lax.while_loop(cond, poll, jnp.int32(0))

        i0 = wid * _I_PER_W
        tok0 = i0 * _NJ
        bufs = ((idxv0, rows0, gs0, is0), (idxv1, rows1, gs1, is1))

        def fire_idx(g, b):
            pltpu.async_copy(
                idx_hbm.at[pl.ds(tok0 + g * _NJ, 256)], bufs[b][0],
                bufs[b][3])

        def wait_idx(b):
            pltpu.make_async_copy(
                idx_hbm.at[pl.ds(0, 256)], bufs[b][0], bufs[b][3]).wait()

        def fire_gathers(b):
            iv, rv, gs, _ = bufs[b]
            pltpu.async_copy(
                stage_hbm.at[iv.at[pl.ds(0, 128)]],
                rv.at[pl.ds(0, 128)], gs)
            pltpu.async_copy(
                stage_hbm.at[iv.at[pl.ds(128, 72)]],
                rv.at[pl.ds(128, 72)], gs)

        def wait_gather(b):
            pltpu.make_async_copy(
                stage_hbm.at[pl.ds(0, _NJ)], bufs[b][1], bufs[b][2]).wait()

        def compact(b):
            rv = bufs[b][1]

            def crow(r, c2):
                for u in range(8):
                    rr = r * 8 + u
                    for j in range(4):
                        sl = pl.ds(j * 16, 16)
                        slab[rr, sl] = rv[rr, sl] * SCALE
                return c2

            lax.fori_loop(0, _NJ // 8, crow, 0)

        def fire_wb(g):
            pltpu.async_copy(slab, out_hbm.at[i0 + g], wsem)

        def wait_wb():
            pltpu.make_async_copy(slab, out_hbm.at[0], wsem).wait()

        fire_idx(0, 0)
        fire_idx(1, 1)
        wait_idx(0)
        fire_gathers(0)
        wait_idx(1)
        fire_gathers(1)

        def step(s, carry):
            for b in range(2):
                g = s * 2 + b
                wait_gather(b)

                @pl.when(g + 2 < _I_PER_W)
                def _():
                    fire_idx(g + 2, b)

                @pl.when(g > 0)
                def _():
                    wait_wb()

                compact(b)

                @pl.when(g + 2 < _I_PER_W)
                def _():
                    wait_idx(b)
                    fire_gathers(b)

                fire_wb(g)
            return carry

        lax.fori_loop(0, _I_PER_W // 2, step, 0)
        wait_wb()


    return k


def kernel(tokens, embedding):
    idx = jnp.pad(tokens.reshape(_NI * _NJ).astype(jnp.int32), (0, 256))
    out, _ = _make_kernel()(idx, embedding.T)
    return out


# speedup vs baseline: 1.2993x; 1.2993x over previous
import functools

import jax
import jax.numpy as jnp
from jax import lax
from jax.experimental import pallas as pl
from jax.experimental.pallas import tpu as pltpu
from jax.experimental.pallas import tpu_sc as plsc

D = 64
SCALE = 8.0

_NC = 2
_NS = 16
_NW = _NC * _NS

_NI = 4096
_NJ = 200
_V = 1000000
_KV = (_V + 127) // 128
_VP = _KV * 128
_FLAGF = 1325.0625

_I_PER_W = _NI // _NW


def _make_kernel():
    mesh = plsc.VectorSubcoreMesh(core_axis_name="c", subcore_axis_name="s")

    @functools.partial(
        pl.kernel,
        mesh=mesh,
        out_type=(
            jax.ShapeDtypeStruct((_NI, _NJ, D), jnp.float32),
            jax.ShapeDtypeStruct((_VP + _NW, 128), jnp.float32),
        ),
        scratch_types=[
            pltpu.VMEM((64, 128), jnp.float32),
            pltpu.VMEM((64, 128), jnp.float32),
            pltpu.VMEM((64, 128), jnp.float32),
            pltpu.VMEM((64, 128), jnp.float32),
            pltpu.VMEM((_NJ, 128), jnp.float32),
            pltpu.VMEM((_NJ, 128), jnp.float32),
            pltpu.VMEM((_NJ, D), jnp.float32),
            pltpu.VMEM((256,), jnp.int32),
            pltpu.VMEM((256,), jnp.int32),
            pltpu.VMEM((_NW, 128), jnp.float32),
            pltpu.VMEM((128,), jnp.float32),
            pltpu.VMEM((272,), jnp.float32),
            pltpu.VMEM((272,), jnp.float32),
            pltpu.VMEM((272,), jnp.float32),
            pltpu.VMEM((272,), jnp.float32),
            pltpu.SemaphoreType.DMA,
            pltpu.SemaphoreType.DMA,
            pltpu.SemaphoreType.DMA,
            pltpu.SemaphoreType.DMA,
            pltpu.SemaphoreType.DMA,
            pltpu.SemaphoreType.DMA,
            pltpu.SemaphoreType.DMA,
            pltpu.SemaphoreType.DMA,
            pltpu.SemaphoreType.DMA,
        ],
        compiler_params=pltpu.CompilerParams(
            use_tc_tiling_on_sc=True, needs_layout_passes=False),
    )
    def k(idx_hbm, tabt_hbm, out_hbm, stage_hbm,
          tbuf0, tbuf1, trow0, trow1, rows0, rows1, slab, idxv0, idxv1,
          flagv, fbuf, bnc0, bnc1, bnc2, bnc3, ts0, ts1, ws0, ws1, gs0, gs1, is0, is1,
          wsem):
        wid = lax.axis_index("s") * _NC + lax.axis_index("c")
        tb = (tbuf0, tbuf1)
        tr = (trow0, trow1)
        tsem = (ts0, ts1)
        wsm = (ws0, ws1)

        for t in range(8):
            fbuf[pl.ds(t * 16, 16)] = jnp.zeros((16,), jnp.float32)
        pltpu.sync_copy(fbuf, stage_hbm.at[_VP + wid])

        kv0 = (wid * _KV) // _NW
        kv1 = ((wid + 1) * _KV) // _NW

        def fire_tiles(kv, b):
            for kd in range(8):
                pltpu.async_copy(
                    tabt_hbm.at[pl.ds(kd * 8, 8), pl.ds(kv * 128, 128)],
                    tb[b].at[pl.ds(kd * 8, 8)],
                    tsem[b],
                )

        def wait_tiles(b):
            pltpu.make_async_copy(
                stage_hbm.at[pl.ds(0, 64)], tb[b], tsem[b]).wait()

        def wait_stage_wb(h):
            pltpu.make_async_copy(
                tr[h], stage_hbm.at[pl.ds(0, 64)], wsm[h]).wait()

        def pcol(kv, b):
            @pl.when(kv + 1 < kv1)
            def _():
                fire_tiles(kv + 1, 1 - b)

            wait_tiles(b)
            iota17 = lax.iota(jnp.int32, 16) * 17

            for h in range(2):
                @pl.when(kv - kv0 >= 1)
                def _():
                    wait_stage_wb(h)

                def trans(v4, c2):
                    vb = v4 + h * 4
                    for db in range(4):
                        bnc = (bnc0, bnc1, bnc2, bnc3)[db]
                        for i in range(16):
                            bnc[pl.ds(i * 17, 16)] = (
                                tb[b][db * 16 + i, pl.ds(vb * 16, 16)])
                    for db in range(4):
                        bnc = (bnc0, bnc1, bnc2, bnc3)[db]
                        for c in range(16):
                            vals = plsc.load_gather(bnc, [iota17 + c])
                            tr[h][v4 * 16 + c, pl.ds(db * 16, 16)] = vals
                    return c2

                lax.fori_loop(0, 4, trans, 0)
                pltpu.async_copy(
                    tr[h], stage_hbm.at[pl.ds(kv * 128 + h * 64, 64)],
                    wsm[h])

        fire_tiles(kv0, 0)

        def pstep(s, carry):
            for par in range(2):
                kv = kv0 + s * 2 + par

                @pl.when(kv < kv1)
                def _():
                    pcol(kv, par)

            return carry

        lax.fori_loop(0, (_KV // _NW) // 2 + 1, pstep, 0)

        wait_stage_wb(0)
        wait_stage_wb(1)

        for t in range(8):
            fbuf[pl.ds(t * 16, 16)] = jnp.full((16,), _FLAGF, jnp.float32)
        pltpu.sync_copy(fbuf, stage_hbm.at[_VP + wid])

        def cond(c):
            return c != _NW * 16

        def poll(c):
            pltpu.sync_copy(stage_hbm.at[pl.ds(_VP, _NW)], flagv)
            n = jnp.zeros((16,), jnp.int32)
            for t in range(_NW):
                n = n + jnp.where(flagv[t, pl.ds(0, 16)] == _FLAGF, 1, 0)
            return jnp.sum(n)

        lax.while_loop(cond, poll, jnp.int32(0))

        i0 = wid * _I_PER_W
        tok0 = i0 * _NJ
        bufs = ((idxv0, rows0, gs0, is0), (idxv1, rows1, gs1, is1))

        def fire_idx(g, b):
            pltpu.async_copy(
                idx_hbm.at[pl.ds(tok0 + g * _NJ, 256)], bufs[b][0],
                bufs[b][3])

        def wait_idx(b):
            pltpu.make_async_copy(
                idx_hbm.at[pl.ds(0, 256)], bufs[b][0], bufs[b][3]).wait()

        def fire_gathers(b):
            iv, rv, gs, _ = bufs[b]
            pltpu.async_copy(
                stage_hbm.at[iv.at[pl.ds(0, 128)]],
                rv.at[pl.ds(0, 128)], gs)
            pltpu.async_copy(
                stage_hbm.at[iv.at[pl.ds(128, 72)]],
                rv.at[pl.ds(128, 72)], gs)

        def wait_gather(b):
            pltpu.make_async_copy(
                stage_hbm.at[pl.ds(0, _NJ)], bufs[b][1], bufs[b][2]).wait()

        def compact(b):
            rv = bufs[b][1]

            def crow(r, c2):
                for u in range(8):
                    rr = r * 8 + u
                    for j in range(4):
                        sl = pl.ds(j * 16, 16)
                        slab[rr, sl] = rv[rr, sl] * SCALE
                return c2

            lax.fori_loop(0, _NJ // 8, crow, 0)

        def fire_wb(g):
            pltpu.async_copy(slab, out_hbm.at[i0 + g], wsem)

        def wait_wb():
            pltpu.make_async_copy(slab, out_hbm.at[0], wsem).wait()

        fire_idx(0, 0)
        fire_idx(1, 1)
        wait_idx(0)
        fire_gathers(0)
        wait_idx(1)
        fire_gathers(1)

        def step(s, carry):
            for b in range(2):
                g = s * 2 + b
                wait_gather(b)

                @pl.when(g + 2 < _I_PER_W)
                def _():
                    fire_idx(g + 2, b)

                @pl.when(g > 0)
                def _():
                    wait_wb()

                compact(b)

                @pl.when(g + 2 < _I_PER_W)
                def _():
                    wait_idx(b)
                    fire_gathers(b)

                fire_wb(g)
            return carry

        lax.fori_loop(0, _I_PER_W // 2, step, 0)
        wait_wb()


    return k


def kernel(tokens, embedding):
    idx = jnp.pad(tokens.reshape(_NI * _NJ).astype(jnp.int32), (0, 256))
    out, _ = _make_kernel()(idx, embedding.T)
    return out


# speedup vs baseline: 1.4915x; 1.1479x over previous
import functools

import jax
import jax.numpy as jnp
from jax import lax
from jax.experimental import pallas as pl
from jax.experimental.pallas import tpu as pltpu
from jax.experimental.pallas import tpu_sc as plsc

D = 64
SCALE = 8.0

_NC = 2
_NS = 16
_NW = _NC * _NS

_NI = 4096
_NJ = 200
_V = 1000000
_KV = (_V + 127) // 128
_VP = _KV * 128
_FLAGF = 1325.0625

_I_PER_W = _NI // _NW


def _make_kernel():
    mesh = plsc.VectorSubcoreMesh(core_axis_name="c", subcore_axis_name="s")

    @functools.partial(
        pl.kernel,
        mesh=mesh,
        out_type=(
            jax.ShapeDtypeStruct((_NI, _NJ, D), jnp.float32),
            jax.ShapeDtypeStruct((_VP + _NW, 128), jnp.float32),
        ),
        scratch_types=[
            pltpu.VMEM((64, 128), jnp.float32),
            pltpu.VMEM((64, 128), jnp.float32),
            pltpu.VMEM((64, 128), jnp.float32),
            pltpu.VMEM((64, 128), jnp.float32),
            pltpu.VMEM((_NJ, 128), jnp.float32),
            pltpu.VMEM((_NJ, 128), jnp.float32),
            pltpu.VMEM((_NJ, D), jnp.float32),
            pltpu.VMEM((256,), jnp.int32),
            pltpu.VMEM((256,), jnp.int32),
            pltpu.VMEM((_NW, 128), jnp.float32),
            pltpu.VMEM((128,), jnp.float32),
            pltpu.VMEM((16 * 272,), jnp.float32),
            pltpu.SemaphoreType.DMA,
            pltpu.SemaphoreType.DMA,
            pltpu.SemaphoreType.DMA,
            pltpu.SemaphoreType.DMA,
            pltpu.SemaphoreType.DMA,
            pltpu.SemaphoreType.DMA,
            pltpu.SemaphoreType.DMA,
            pltpu.SemaphoreType.DMA,
            pltpu.SemaphoreType.DMA,
        ],
        compiler_params=pltpu.CompilerParams(
            use_tc_tiling_on_sc=True, needs_layout_passes=False),
    )
    def k(idx_hbm, tabt_hbm, out_hbm, stage_hbm,
          tbuf0, tbuf1, trow0, trow1, rows0, rows1, slab, idxv0, idxv1,
          flagv, fbuf, bnc, ts0, ts1, ws0, ws1, gs0, gs1, is0, is1,
          wsem):
        wid = lax.axis_index("s") * _NC + lax.axis_index("c")
        tb = (tbuf0, tbuf1)
        tr = (trow0, trow1)
        tsem = (ts0, ts1)
        wsm = (ws0, ws1)

        for t in range(8):
            fbuf[pl.ds(t * 16, 16)] = jnp.zeros((16,), jnp.float32)
        pltpu.sync_copy(fbuf, stage_hbm.at[_VP + wid])

        kv0 = (wid * _KV) // _NW
        kv1 = ((wid + 1) * _KV) // _NW

        def fire_tiles(kv, b):
            for kd in range(8):
                pltpu.async_copy(
                    tabt_hbm.at[pl.ds(kd * 8, 8), pl.ds(kv * 128, 128)],
                    tb[b].at[pl.ds(kd * 8, 8)],
                    tsem[b],
                )

        def wait_tiles(b):
            pltpu.make_async_copy(
                stage_hbm.at[pl.ds(0, 64)], tb[b], tsem[b]).wait()

        def wait_stage_wb(h):
            pltpu.make_async_copy(
                tr[h], stage_hbm.at[pl.ds(0, 64)], wsm[h]).wait()

        def pcol(kv, b):
            @pl.when(kv + 1 < kv1)
            def _():
                fire_tiles(kv + 1, 1 - b)

            wait_tiles(b)
            iota17 = lax.iota(jnp.int32, 16) * 17

            for h in range(2):
                @pl.when(kv - kv0 >= 1)
                def _():
                    wait_stage_wb(h)

                @plsc.parallel_loop(0, 4)
                def _trans(v4):
                    vb = v4 + h * 4
                    for db in range(4):
                        r0 = (v4 * 4 + db) * 272
                        for i in range(16):
                            bnc[pl.ds(r0 + i * 17, 16)] = (
                                tb[b][db * 16 + i, pl.ds(vb * 16, 16)])
                        for c in range(16):
                            vals = plsc.load_gather(bnc, [iota17 + r0 + c])
                            tr[h][v4 * 16 + c, pl.ds(db * 16, 16)] = vals
                pltpu.async_copy(
                    tr[h], stage_hbm.at[pl.ds(kv * 128 + h * 64, 64)],
                    wsm[h])

        fire_tiles(kv0, 0)

        def pstep(s, carry):
            for par in range(2):
                kv = kv0 + s * 2 + par

                @pl.when(kv < kv1)
                def _():
                    pcol(kv, par)

            return carry

        lax.fori_loop(0, (_KV // _NW) // 2 + 1, pstep, 0)

        wait_stage_wb(0)
        wait_stage_wb(1)

        for t in range(8):
            fbuf[pl.ds(t * 16, 16)] = jnp.full((16,), _FLAGF, jnp.float32)
        pltpu.sync_copy(fbuf, stage_hbm.at[_VP + wid])

        def cond(c):
            return c != _NW * 16

        def poll(c):
            pltpu.sync_copy(stage_hbm.at[pl.ds(_VP, _NW)], flagv)
            n = jnp.zeros((16,), jnp.int32)
            for t in range(_NW):
                n = n + jnp.where(flagv[t, pl.ds(0, 16)] == _FLAGF, 1, 0)
            return jnp.sum(n)

        lax.while_loop(cond, poll, jnp.int32(0))

        i0 = wid * _I_PER_W
        tok0 = i0 * _NJ
        bufs = ((idxv0, rows0, gs0, is0), (idxv1, rows1, gs1, is1))

        def fire_idx(g, b):
            pltpu.async_copy(
                idx_hbm.at[pl.ds(tok0 + g * _NJ, 256)], bufs[b][0],
                bufs[b][3])

        def wait_idx(b):
            pltpu.make_async_copy(
                idx_hbm.at[pl.ds(0, 256)], bufs[b][0], bufs[b][3]).wait()

        def fire_gathers(b):
            iv, rv, gs, _ = bufs[b]
            pltpu.async_copy(
                stage_hbm.at[iv.at[pl.ds(0, 128)]],
                rv.at[pl.ds(0, 128)], gs)
            pltpu.async_copy(
                stage_hbm.at[iv.at[pl.ds(128, 72)]],
                rv.at[pl.ds(128, 72)], gs)

        def wait_gather(b):
            pltpu.make_async_copy(
                stage_hbm.at[pl.ds(0, _NJ)], bufs[b][1], bufs[b][2]).wait()

        def compact(b):
            rv = bufs[b][1]

            def crow(r, c2):
                for u in range(8):
                    rr = r * 8 + u
                    for j in range(4):
                        sl = pl.ds(j * 16, 16)
                        slab[rr, sl] = rv[rr, sl] * SCALE
                return c2

            lax.fori_loop(0, _NJ // 8, crow, 0)

        def fire_wb(g):
            pltpu.async_copy(slab, out_hbm.at[i0 + g], wsem)

        def wait_wb():
            pltpu.make_async_copy(slab, out_hbm.at[0], wsem).wait()

        fire_idx(0, 0)
        fire_idx(1, 1)
        wait_idx(0)
        fire_gathers(0)
        wait_idx(1)
        fire_gathers(1)

        def step(s, carry):
            for b in range(2):
                g = s * 2 + b
                wait_gather(b)

                @pl.when(g + 2 < _I_PER_W)
                def _():
                    fire_idx(g + 2, b)

                @pl.when(g > 0)
                def _():
                    wait_wb()

                compact(b)

                @pl.when(g + 2 < _I_PER_W)
                def _():
                    wait_idx(b)
                    fire_gathers(b)

                fire_wb(g)
            return carry

        lax.fori_loop(0, _I_PER_W // 2, step, 0)
        wait_wb()


    return k


def kernel(tokens, embedding):
    idx = jnp.pad(tokens.reshape(_NI * _NJ).astype(jnp.int32), (0, 256))
    out, _ = _make_kernel()(idx, embedding.T)
    return out


# speedup vs baseline: 1.6852x; 1.1299x over previous
import functools

import jax
import jax.numpy as jnp
from jax import lax
from jax.experimental import pallas as pl
from jax.experimental.pallas import tpu as pltpu
from jax.experimental.pallas import tpu_sc as plsc

D = 64
SCALE = 8.0

_NC = 2
_NS = 16
_NW = _NC * _NS

_IDXW = 128
_CROWS = 5
_CHUNK = _CROWS * _IDXW


def _make_sc_kernel(B):
    rows_per_w = B // _IDXW // _NW
    nchunks = rows_per_w // _CROWS
    mesh = plsc.VectorSubcoreMesh(core_axis_name="c", subcore_axis_name="s")

    @functools.partial(
        pl.kernel,
        mesh=mesh,
        out_type=jax.ShapeDtypeStruct((B, D), jnp.float32),
        scratch_types=[
            pltpu.VMEM((rows_per_w, _IDXW), jnp.int32),
            pltpu.VMEM((2, _CHUNK, D), jnp.float32),
            pltpu.SemaphoreType.DMA,
            pltpu.SemaphoreType.DMA,
            pltpu.SemaphoreType.DMA,
            pltpu.SemaphoreType.DMA,
        ],
        compiler_params=pltpu.CompilerParams(use_tc_tiling_on_sc=False),
    )
    def k(idx_hbm, table_hbm, out_hbm, idx_v, rows_v, g0, g1, w0, w1):
        wid = lax.axis_index("s") * _NC + lax.axis_index("c")
        row0 = wid * rows_per_w
        out0 = row0 * _IDXW
        gsem = (g0, g1)
        wsem = (w0, w1)

        def fire(g, b):
            for j in range(_CROWS):
                pltpu.async_copy(
                    table_hbm.at[idx_v.at[g * _CROWS + j]],
                    rows_v.at[b].at[pl.ds(j * _IDXW, _IDXW)],
                    gsem[b],
                )

        def wait_gather(b):
            pltpu.make_async_copy(
                out_hbm.at[pl.ds(0, _CHUNK)], rows_v.at[b], gsem[b]
            ).wait()

        def scale(b):
            def body(i, c):
                for u in range(8):
                    r = i * 8 + u
                    for j in range(D // 16):
                        sl = pl.ds(j * 16, 16)
                        rows_v[b, r, sl] = rows_v[b, r, sl] * SCALE
                return c

            lax.fori_loop(0, _CHUNK // 8, body, 0)

        def fire_wb(g, b):
            pltpu.async_copy(
                rows_v.at[b], out_hbm.at[pl.ds(out0 + g * _CHUNK, _CHUNK)],
                wsem[b],
            )

        def wait_wb(b):
            pltpu.make_async_copy(
                rows_v.at[b], out_hbm.at[pl.ds(0, _CHUNK)], wsem[b]
            ).wait()

        pltpu.sync_copy(idx_hbm.at[pl.ds(row0, rows_per_w)], idx_v)

        fire(0, 0)
        fire(1, 1)
        wait_gather(0)
        scale(0)
        fire_wb(0, 0)

        def outer(s, carry):
            for par in range(2):
                g = 1 + s * 2 + par
                b = (1 + par) % 2
                nb = 1 - b
                wait_wb(nb)
                fire(g + 1, nb)
                wait_gather(b)
                scale(b)
                fire_wb(g, b)
            return carry

        lax.fori_loop(0, (nchunks - 2) // 2, outer, 0)

        gl = nchunks - 1
        bl = gl % 2
        wait_gather(bl)
        scale(bl)
        fire_wb(gl, bl)
        wait_wb(0)
        wait_wb(1)

    return k


def kernel(tokens, embedding):
    B = tokens.shape[0] * tokens.shape[1]
    idx = tokens.reshape(B // _IDXW, _IDXW).astype(jnp.int32)
    out = _make_sc_kernel(B)(idx, embedding)
    return out.reshape(*tokens.shape, D)
